# baseline (speedup 1.0000x reference)
import jax
import jax.numpy as jnp
from jax import lax
from jax.experimental import pallas as pl
from jax.experimental.pallas import tpu as pltpu
from jax.experimental.pallas import tpu_sc as plsc

N = 10000
E = 320000
D = 128
DEG = 16
ALPHA = 0.05
BETA = 1.0

NW = 32
RPW = 320
NPAD = NW * RPW
C = 128
GC = 8
B = GC * C
EPAD = E + B
RP_LEN = NPAD + 336


def _spmm_body(feat_in, emb_in, src_p, w_p, rowptr, winp,
               feat_out, emb_out,
               rp_v, srcb, wch, winb, g0, g1, g2, g3, acc,
               sem_st, sem_g0, sem_g1, sem_g2, sem_g3):
    cid = lax.axis_index("c")
    sid = lax.axis_index("s")
    wid = sid * 2 + cid
    row_start = wid * RPW

    pltpu.sync_copy(rowptr.at[pl.ds(row_start, 336)], rp_v)
    start = rp_v[pl.ds(0, 16)][0]
    end = rp_v[pl.ds(RPW, 16)][0]

    zero16 = jnp.zeros((16,), jnp.float32)

    @plsc.parallel_loop(0, RPW, unroll=2)
    def zero_body(r):
        for j in range(8):
            acc[r, pl.ds(j * 16, 16)] = zero16

    a0 = (start // B) * B
    nc = (end - a0 + C - 1) // C
    ng = (nc + GC - 1) // GC

    gbufs = (g0, g1, g2, g3)
    gsems = (sem_g0, sem_g1, sem_g2, sem_g3)

    def issue_gather(ci, b):
        @pl.when(ci < nc)
        def _():
            pltpu.async_copy(feat_in.at[srcb.at[b % GC]],
                             gbufs[b % 4], gsems[b % 4])

    def do_chunk(ci, b, wv16, gbase):
        ck_start = a0 + ci * C
        ck_end = ck_start + C
        gq = gbufs[b % 4]

        @pl.when(ci < nc)
        def _():
            pltpu.make_async_copy(feat_in.at[srcb.at[b % GC]],
                                  gbufs[b % 4], gsems[b % 4]).wait()
            r_lo = jnp.clip(wv16[2 * b] - row_start, 0, RPW - 1)
            r_hi = jnp.clip(wv16[2 * b + 1] - row_start, 0, RPW - 1)

            @plsc.parallel_loop(r_lo, r_hi + 1)
            def row_body(r):
                rv = rp_v[pl.ds(r, 16)]
                s_c = jnp.maximum(rv[0], ck_start)
                t_c = jnp.minimum(rv[1], ck_end)

                def ebody(e, v):
                    lo = e - ck_start
                    w = wch[pl.ds(e - gbase, 16)][0]
                    return tuple(
                        v[j] + w * gq[lo, pl.ds(j * 16, 16)]
                        for j in range(8))

                vs = lax.fori_loop(s_c, t_c, ebody, (zero16,) * 8)

                @pl.when(t_c > s_c)
                def _():
                    for j in range(8):
                        plsc.addupdate(acc.at[r, pl.ds(j * 16, 16)], vs[j])

    def group_body(gi, carry):
        grow = pl.multiple_of(a0 // C + gi * GC, 8)
        cp_s = pltpu.async_copy(
            src_p.at[pl.ds(grow, GC)], srcb, sem_st)
        gbase = a0 + gi * B
        cp_w = pltpu.async_copy(
            w_p.at[pl.ds(gbase, B)], wch.at[pl.ds(0, B)], sem_st)
        cp_b = pltpu.async_copy(
            winp.at[pl.ds(2 * grow, 16)], winb, sem_st)
        cp_s.wait()
        cp_w.wait()
        cp_b.wait()
        wv16 = winb[pl.ds(0, 16)]

        issue_gather(gi * GC + 0, 0)
        issue_gather(gi * GC + 1, 1)
        for b in range(GC):
            ci = gi * GC + b
            if b + 2 < GC:
                issue_gather(ci + 2, b + 2)
            do_chunk(ci, b, wv16, gbase)
        return 0

    lax.fori_loop(0, ng, group_body, 0)

    pltpu.sync_copy(acc, feat_out.at[pl.ds(row_start, RPW)])

    for (buf, r0_, nr) in ((g0, 0, 128), (g1, 128, 128), (g2, 256, 64)):
        pltpu.sync_copy(emb_in.at[pl.ds(row_start + r0_, nr)],
                        buf.at[pl.ds(0, nr)])

        @plsc.parallel_loop(0, nr, unroll=2)
        def add_body(r):
            for j in range(8):
                buf[r, pl.ds(j * 16, 16)] = (
                    buf[r, pl.ds(j * 16, 16)]
                    + acc[r0_ + r, pl.ds(j * 16, 16)])

        pltpu.sync_copy(buf.at[pl.ds(0, nr)],
                        emb_out.at[pl.ds(row_start + r0_, nr)])


_spmm_step = pl.kernel(
    _spmm_body,
    out_type=(
        jax.ShapeDtypeStruct((NPAD, D), jnp.float32),
        jax.ShapeDtypeStruct((NPAD, D), jnp.float32),
    ),
    mesh=plsc.VectorSubcoreMesh(core_axis_name="c", subcore_axis_name="s"),
    scratch_types=[
        pltpu.VMEM((336,), jnp.int32),
        pltpu.VMEM((GC, C), jnp.int32),
        pltpu.VMEM((B + 16,), jnp.float32),
        pltpu.VMEM((16,), jnp.int32),
        pltpu.VMEM((C, D), jnp.float32),
        pltpu.VMEM((C, D), jnp.float32),
        pltpu.VMEM((C, D), jnp.float32),
        pltpu.VMEM((C, D), jnp.float32),
        pltpu.VMEM((RPW, D), jnp.float32),
        pltpu.SemaphoreType.DMA,
        pltpu.SemaphoreType.DMA,
        pltpu.SemaphoreType.DMA,
        pltpu.SemaphoreType.DMA,
        pltpu.SemaphoreType.DMA,
    ],
)


def kernel(input, edge_index, edge_weight):
    dst = edge_index[0]
    src = edge_index[1]
    dst_s, src_s, w_s = lax.sort(
        (dst, src, edge_weight * ALPHA), num_keys=1)

    pad = EPAD - E
    src_p = jnp.concatenate(
        [src_s, jnp.zeros((pad,), jnp.int32)]).reshape(EPAD // C, C)
    w_p = jnp.concatenate([w_s, jnp.zeros((pad,), jnp.float32)])

    rowptr = jnp.searchsorted(
        dst_s, jnp.arange(NPAD + 1, dtype=jnp.int32)).astype(jnp.int32)
    rowptr = jnp.concatenate(
        [rowptr, jnp.full((RP_LEN - NPAD - 1,), E, jnp.int32)])

    dst_pad = jnp.concatenate(
        [dst_s, jnp.full((pad,), NPAD - 1, jnp.int32)])
    wlo = dst_pad[::C]
    whi = dst_pad[C - 1::C]
    winp = jnp.stack([wlo, whi], axis=1).reshape(2 * (EPAD // C))
    winp = jnp.concatenate([winp, jnp.zeros((24,), jnp.int32)])

    feat = jnp.pad(input, ((0, NPAD - N), (0, 0)))
    emb = feat
    for _ in range(DEG):
        feat, emb = _spmm_step(feat, emb, src_p, w_p, rowptr, winp)

    out = emb[:N] / DEG
    return BETA * out + (1.0 - BETA) * input
